# baseline (speedup 1.0000x reference)
import functools

import jax
import jax.numpy as jnp
from jax import lax
from jax.experimental import pallas as pl
from jax.experimental.pallas import tpu as pltpu
from jax.experimental.pallas import tpu_sc as plsc

_D = 64
_NC = 2
_NS = 16
_NW = _NC * _NS
_GB = 128
_GPW = 200


def _emb_body(table_hbm, idx_hbm, out_hbm,
              idx_v, g0, g1, t0, t1, gsem0, gsem1, wsem0, wsem1):
    wid = lax.axis_index("s") * _NC + lax.axis_index("c")
    base_g = wid * _GPW

    pltpu.sync_copy(idx_hbm.at[pl.ds(base_g * _GB, _GPW * _GB)], idx_v)

    gbuf = (g0, g1)
    tbuf = (t0, t1)
    gsem = (gsem0, gsem1)
    wsem = (wsem0, wsem1)
    eiota = lax.iota(jnp.int32, 16)
    ebs = [(eiota + k * 16) >> 3 for k in range(4)]
    eis = [(eiota + k * 16) & 7 for k in range(4)]

    def gather(lg, b):
        return pltpu.make_async_copy(
            table_hbm.at[idx_v.at[pl.ds(lg * _GB, _GB)]], gbuf[b], gsem[b])

    def writeback(lg, b):
        gid = base_g + lg
        s = gid // 32
        bb = gid - s * 32
        return pltpu.make_async_copy(
            tbuf[b].at[:, :, pl.ds(0, _GB)], out_hbm.at[s, :, bb], wsem[b])

    def transpose(b):
        src, dst = gbuf[b], tbuf[b]

        @plsc.parallel_loop(0, _GB, unroll=8)
        def row(j):
            jv = jnp.full((16,), 0, jnp.int32) + j
            for k in range(4):
                v = src[j, pl.ds(k * 16, 16)]
                plsc.store_scatter(dst, [ebs[k], eis[k], jv], v)

    def step(lg, b, *, first, last):
        if not first:
            writeback(lg - 2, b).wait()
        gather(lg, b).wait()
        transpose(b)
        writeback(lg, b).start()
        if not last:
            gather(lg + 2, b).start()

    gather(0, 0).start()
    gather(1, 1).start()
    step(0, 0, first=True, last=False)
    step(1, 1, first=True, last=False)

    def body(i, carry):
        lg = i * 2
        step(lg, 0, first=False, last=False)
        step(lg + 1, 1, first=False, last=False)
        return carry

    lax.fori_loop(1, _GPW // 2 - 1, body, 0)

    step(_GPW - 2, 0, first=False, last=True)
    step(_GPW - 1, 1, first=False, last=True)

    writeback(_GPW - 2, 0).wait()
    writeback(_GPW - 1, 1).wait()


def _depad_body(in_ref, out_ref):
    x = in_ref[...]
    a = x.reshape(out_ref.shape[0], 2, _D)
    out_ref[...] = jnp.concatenate([a[:, 0, :], a[:, 1, :]], axis=-1)


def _depad(table):
    vocab = table.shape[0]
    blk = 8000
    return pl.pallas_call(
        _depad_body,
        grid=(vocab // blk,),
        in_specs=[pl.BlockSpec((blk, _D), lambda i: (i, 0))],
        out_specs=pl.BlockSpec((blk // 2, 2 * _D), lambda i: (i, 0)),
        out_shape=jax.ShapeDtypeStruct((vocab // 2, 2 * _D), jnp.float32),
    )(table)


def kernel(tokenized_sentence, table):
    batch, seq = tokenized_sentence.shape
    idx = tokenized_sentence.T.reshape(batch * seq).astype(jnp.int32)
    table_lin = _depad(table).reshape(table.shape)

    mesh = plsc.VectorSubcoreMesh(core_axis_name="c", subcore_axis_name="s")
    k = pl.kernel(
        _emb_body,
        mesh=mesh,
        out_type=jax.ShapeDtypeStruct((seq, _D // 8, batch // _GB, 8, _GB),
                                      jnp.float32),
        scratch_types=[
            pltpu.VMEM((_GPW * _GB,), jnp.int32),
            pltpu.VMEM((_GB, _D), jnp.float32),
            pltpu.VMEM((_GB, _D), jnp.float32),
            pltpu.VMEM((_D // 8, 8, _GB + 1), jnp.float32),
            pltpu.VMEM((_D // 8, 8, _GB + 1), jnp.float32),
            pltpu.SemaphoreType.DMA,
            pltpu.SemaphoreType.DMA,
            pltpu.SemaphoreType.DMA,
            pltpu.SemaphoreType.DMA,
        ],
        compiler_params=pltpu.CompilerParams(use_tc_tiling_on_sc=False,
                                             needs_layout_passes=False),
    )
    out5 = k(table_lin, idx)
    return out5.transpose(2, 4, 0, 1, 3).reshape(batch, seq, _D)


# speedup vs baseline: 1.1435x; 1.1435x over previous
import functools

import jax
import jax.numpy as jnp
from jax import lax
from jax.experimental import pallas as pl
from jax.experimental.pallas import tpu as pltpu
from jax.experimental.pallas import tpu_sc as plsc

_D = 64
_NC = 2
_NS = 16
_NW = _NC * _NS
_GB = 128
_GPW = 200


def _emb_body(table_hbm, idx_hbm, out_hbm,
              idx_v, g0, g1, t0, t1, gsem0, gsem1, wsem0, wsem1):
    wid = lax.axis_index("s") * _NC + lax.axis_index("c")
    base_g = wid * _GPW

    pltpu.sync_copy(idx_hbm.at[pl.ds(base_g * _GB, _GPW * _GB)], idx_v)

    gbuf = (g0, g1)
    tbuf = (t0, t1)
    gsem = (gsem0, gsem1)
    wsem = (wsem0, wsem1)
    eiota = lax.iota(jnp.int32, 16)
    ebs = [(eiota + k * 16) >> 3 for k in range(4)]
    eis = [(eiota + k * 16) & 7 for k in range(4)]

    def gather(lg, b):
        return pltpu.make_async_copy(
            table_hbm.at[idx_v.at[pl.ds(lg * _GB, _GB)]], gbuf[b], gsem[b])

    def writeback(lg, b):
        gid = base_g + lg
        s = gid // 32
        bb = gid - s * 32
        return pltpu.make_async_copy(
            tbuf[b].at[:, :, pl.ds(0, _GB)], out_hbm.at[s, :, bb], wsem[b])

    def transpose(b):
        src, dst = gbuf[b], tbuf[b]

        @plsc.parallel_loop(0, _GB, unroll=8)
        def row(j):
            jv = jnp.full((16,), 0, jnp.int32) + j
            for k in range(4):
                v = src[j, pl.ds(k * 16, 16)]
                plsc.store_scatter(dst, [ebs[k], eis[k], jv], v)

    def step(lg, b, *, first, last):
        if not first:
            writeback(lg - 2, b).wait()
        gather(lg, b).wait()
        transpose(b)
        writeback(lg, b).start()
        if not last:
            gather(lg + 2, b).start()

    gather(0, 0).start()
    gather(1, 1).start()
    step(0, 0, first=True, last=False)
    step(1, 1, first=True, last=False)

    def body(i, carry):
        lg = i * 2
        step(lg, 0, first=False, last=False)
        step(lg + 1, 1, first=False, last=False)
        return carry

    lax.fori_loop(1, _GPW // 2 - 1, body, 0)

    step(_GPW - 2, 0, first=False, last=True)
    step(_GPW - 1, 1, first=False, last=True)

    writeback(_GPW - 2, 0).wait()
    writeback(_GPW - 1, 1).wait()


def kernel(tokenized_sentence, table):
    batch, seq = tokenized_sentence.shape
    idx = tokenized_sentence.T.reshape(batch * seq).astype(jnp.int32)

    mesh = plsc.VectorSubcoreMesh(core_axis_name="c", subcore_axis_name="s")
    k = pl.kernel(
        _emb_body,
        mesh=mesh,
        out_type=jax.ShapeDtypeStruct((seq, _D // 8, batch // _GB, 8, _GB),
                                      jnp.float32),
        scratch_types=[
            pltpu.VMEM((_GPW * _GB,), jnp.int32),
            pltpu.VMEM((_GB, _D), jnp.float32),
            pltpu.VMEM((_GB, _D), jnp.float32),
            pltpu.VMEM((_D // 8, 8, _GB + 1), jnp.float32),
            pltpu.VMEM((_D // 8, 8, _GB + 1), jnp.float32),
            pltpu.SemaphoreType.DMA,
            pltpu.SemaphoreType.DMA,
            pltpu.SemaphoreType.DMA,
            pltpu.SemaphoreType.DMA,
        ],
        compiler_params=pltpu.CompilerParams(use_tc_tiling_on_sc=False,
                                             needs_layout_passes=False),
    )
    out5 = k(table, idx)
    return out5.transpose(2, 4, 0, 1, 3).reshape(batch, seq, _D)
